# Initial kernel scaffold; baseline (speedup 1.0000x reference)
import jax
import jax.numpy as jnp
from jax.experimental import pallas as pl


def kernel(real, imag, edge_index, edge_weight, W1, b1, W2, b2, Wc, bc):
    raise NotImplementedError("write your pallas kernel here")


# speedup vs baseline: 1.0022x; 1.0022x over previous
import functools

import jax
import jax.numpy as jnp
import numpy as np
from jax.experimental import pallas as pl
from jax.experimental.pallas import tpu as pltpu

_TWO_PI_Q = 2.0 * np.pi * 0.25


def _coalesce(edge_index, edge_weight, num_nodes):
    r0 = edge_index[0].astype(jnp.int32)
    c0 = edge_index[1].astype(jnp.int32)
    ids = jnp.concatenate([r0 * num_nodes + c0, c0 * num_nodes + r0])
    w_both = jnp.concatenate([edge_weight, edge_weight])
    w_sgn = jnp.concatenate([edge_weight, -edge_weight])
    sk, sw, sd = jax.lax.sort([ids, w_both, w_sgn], num_keys=1)
    head = jnp.concatenate([jnp.ones((1,), jnp.bool_), sk[1:] != sk[:-1]])
    seg = jnp.cumsum(head.astype(jnp.int32)) - 1
    e2 = ids.shape[0]
    w_sym = jax.ops.segment_sum(sw, seg, num_segments=e2,
                                indices_are_sorted=True) * 0.5
    w_dif = jax.ops.segment_sum(sd, seg, num_segments=e2,
                                indices_are_sorted=True)
    key_u = jax.ops.segment_max(sk, seg, num_segments=e2,
                                indices_are_sorted=True)
    key_u = jnp.maximum(key_u, 0)
    r = key_u // num_nodes
    c = key_u % num_nodes
    return r, c, w_sym, w_dif * _TWO_PI_Q


def _norms(r, c, w_sym, theta, num_nodes):
    deg = jax.ops.segment_sum(jnp.abs(w_sym), r, num_segments=num_nodes)
    dinv = jnp.where(deg > 0, jax.lax.rsqrt(jnp.where(deg > 0, deg, 1.0)), 0.0)
    wn = -(dinv[r] * w_sym * dinv[c])
    return wn * jnp.cos(theta), wn * jnp.sin(theta)


def _prop(x, n, r, c, num_nodes):
    return jax.ops.segment_sum(n[:, None] * jnp.take(x, r, axis=0), c,
                               num_segments=num_nodes)


def _layer(xr, xi, r, c, nr, ni, W, b, num_nodes):
    sigs = [xr, xi, xi, xr]
    nrms = [nr, ni, nr, ni]
    outs = [s @ W[0] for s in sigs]
    Tx1 = [_prop(s, n, r, c, num_nodes) for s, n in zip(sigs, nrms)]
    outs = [o + t @ W[1] for o, t in zip(outs, Tx1)]
    Tx2 = [2.0 * _prop(t1, n, r, c, num_nodes) - t0
           for t1, t0, n in zip(Tx1, sigs, nrms)]
    outs = [o + t @ W[2] for o, t in zip(outs, Tx2)]
    out_r = outs[0] - outs[1] + b
    out_i = outs[2] + outs[3] + b
    mask = (out_r >= 0).astype(out_r.dtype)
    return mask * out_r, mask * out_i


def _tail_kernel(z_ref, w_ref, b_ref, zn_ref, out_ref, pred_ref, prob_ref):
    z = z_ref[...]
    logits = jnp.dot(z, w_ref[...], preferred_element_type=jnp.float32)
    logits = logits + b_ref[...]
    m = jnp.max(logits, axis=1, keepdims=True)
    sh = logits - m
    ex = jnp.exp(sh)
    den = jnp.sum(ex, axis=1, keepdims=True)
    out_ref[...] = sh - jnp.log(den)
    prob_ref[...] = ex / den
    pred_ref[...] = jnp.broadcast_to(
        jnp.argmax(logits, axis=1, keepdims=True).astype(jnp.int32),
        logits.shape)
    nrm = jnp.maximum(jnp.sqrt(jnp.sum(z * z, axis=1, keepdims=True)), 1e-12)
    zn_ref[...] = z / nrm


def _tail(z, Wc, bc):
    n, f = z.shape
    l = Wc.shape[0]
    lp = 128
    wp = jnp.full((f, lp), 0.0, jnp.float32).at[:, :l].set(Wc.T)
    bp = jnp.full((1, lp), -1e30, jnp.float32).at[0, :l].set(bc)
    zn, out, pred, prob = pl.pallas_call(
        _tail_kernel,
        out_shape=[
            jax.ShapeDtypeStruct((n, f), jnp.float32),
            jax.ShapeDtypeStruct((n, lp), jnp.float32),
            jax.ShapeDtypeStruct((n, lp), jnp.int32),
            jax.ShapeDtypeStruct((n, lp), jnp.float32),
        ],
    )(z, wp, bp)
    return zn, out[:, :l], pred[:, 0], prob[:, :l]


def kernel(real, imag, edge_index, edge_weight, W1, b1, W2, b2, Wc, bc):
    num_nodes = real.shape[0]
    r, c, w_sym, theta = _coalesce(edge_index, edge_weight, num_nodes)
    nr, ni = _norms(r, c, w_sym, theta, num_nodes)
    xr, xi = _layer(real, imag, r, c, nr, ni, W1, b1, num_nodes)
    xr, xi = _layer(xr, xi, r, c, nr, ni, W2, b2, num_nodes)
    z = jnp.concatenate([xr, xi], axis=-1)
    return _tail(z, Wc, bc)


# speedup vs baseline: 1.6946x; 1.6909x over previous
import functools

import jax
import jax.numpy as jnp
import numpy as np
from jax import lax
from jax.experimental import pallas as pl
from jax.experimental.pallas import tpu as pltpu
from jax.experimental.pallas import tpu_sc as plsc

_TWO_PI_Q = 2.0 * np.pi * 0.25

_N = 10000
_CH = 128
_NTILES = 32
_E2P = 643072
_PER_TILE = _E2P // _NTILES
_NCH = _PER_TILE // _CH
_NPAD = 10240
_ROWS_PER_TILE = _NPAD // 16


@functools.lru_cache(maxsize=None)
def _sc_prop_call(width):
    mesh = plsc.VectorSubcoreMesh(core_axis_name="c", subcore_axis_name="s")
    nvec = width // 16
    cparams = (None if width % 128 == 0
               else pltpu.CompilerParams(use_tc_tiling_on_sc=False))

    @functools.partial(
        pl.kernel,
        mesh=mesh,
        compiler_params=cparams,
        out_type=jax.ShapeDtypeStruct((2, _NPAD, width), jnp.float32),
        scratch_types=[
            pltpu.VMEM((_CH,), jnp.int32),
            pltpu.VMEM((_CH,), jnp.int32),
            pltpu.VMEM((_CH,), jnp.float32),
            pltpu.VMEM((_CH, width), jnp.float32),
            pltpu.VMEM((_CH, width), jnp.float32),
            pltpu.VMEM_SHARED((_NPAD, width), jnp.float32),
            pltpu.SemaphoreType.DMA,
        ],
    )
    def prop(tbl, ridx, cidx, nrm, zeros, out,
             idxr_v, idxc_v, nrm_v, rows_v, sc_v, accum, sem):
        cid = lax.axis_index("c")
        sid = lax.axis_index("s")
        rbase = sid * _ROWS_PER_TILE
        pltpu.sync_copy(zeros.at[pl.ds(rbase, _ROWS_PER_TILE)],
                        accum.at[pl.ds(rbase, _ROWS_PER_TILE)])
        plsc.subcore_barrier()
        ebase = (cid * 16 + sid) * _PER_TILE

        def chunk(k, carry):
            off = ebase + k * _CH
            pltpu.sync_copy(ridx.at[pl.ds(off, _CH)], idxr_v)
            pltpu.sync_copy(nrm.at[pl.ds(off, _CH)], nrm_v)
            pltpu.sync_copy(cidx.at[pl.ds(off, _CH)], idxc_v)
            pltpu.async_copy(tbl.at[idxr_v], rows_v, sem).wait()

            def grp(g, c2):
                n16 = nrm_v[pl.ds(g * 16, 16)]
                for lane in range(16):
                    i = g * 16 + lane
                    s = lax.gather(
                        n16, jnp.full((16, 1), lane, jnp.int32),
                        lax.GatherDimensionNumbers(
                            offset_dims=(), collapsed_slice_dims=(0,),
                            start_index_map=(0,)),
                        (1,), mode=lax.GatherScatterMode.PROMISE_IN_BOUNDS)
                    for j in range(nvec):
                        sc_v[i, pl.ds(j * 16, 16)] = (
                            rows_v[i, pl.ds(j * 16, 16)] * s)
                return c2

            lax.fori_loop(0, _CH // 16, grp, 0)
            pltpu.sync_copy(sc_v, accum.at[idxc_v], add=True)
            return carry

        lax.fori_loop(0, _NCH, chunk, 0)
        plsc.subcore_barrier()
        pltpu.sync_copy(accum.at[pl.ds(rbase, _ROWS_PER_TILE)],
                        out.at[cid, pl.ds(rbase, _ROWS_PER_TILE)])

    return prop


def _sc_prop(tbl, ridx, cidx, nrm, zeros):
    parts = _sc_prop_call(tbl.shape[1])(tbl, ridx, cidx, nrm, zeros)
    return parts[0, :_N] + parts[1, :_N]


def _coalesce(edge_index, edge_weight, num_nodes):
    r0 = edge_index[0].astype(jnp.int32)
    c0 = edge_index[1].astype(jnp.int32)
    ids = jnp.concatenate([r0 * num_nodes + c0, c0 * num_nodes + r0])
    w_both = jnp.concatenate([edge_weight, edge_weight])
    w_sgn = jnp.concatenate([edge_weight, -edge_weight])
    sk, sw, sd = jax.lax.sort([ids, w_both, w_sgn], num_keys=1)
    head = jnp.concatenate([jnp.ones((1,), jnp.bool_), sk[1:] != sk[:-1]])
    seg = jnp.cumsum(head.astype(jnp.int32)) - 1
    e2 = ids.shape[0]
    w_sym = jax.ops.segment_sum(sw, seg, num_segments=e2,
                                indices_are_sorted=True) * 0.5
    w_dif = jax.ops.segment_sum(sd, seg, num_segments=e2,
                                indices_are_sorted=True)
    key_u = jax.ops.segment_max(sk, seg, num_segments=e2,
                                indices_are_sorted=True)
    key_u = jnp.maximum(key_u, 0)
    r = key_u // num_nodes
    c = key_u % num_nodes
    return r, c, w_sym, w_dif * _TWO_PI_Q


def _norms(r, c, w_sym, theta, num_nodes):
    deg = jax.ops.segment_sum(jnp.abs(w_sym), r, num_segments=num_nodes)
    dinv = jnp.where(deg > 0, jax.lax.rsqrt(jnp.where(deg > 0, deg, 1.0)), 0.0)
    wn = -(dinv[r] * w_sym * dinv[c])
    return wn * jnp.cos(theta), wn * jnp.sin(theta)


def _prop(x, n, r, c, num_nodes):
    return jax.ops.segment_sum(n[:, None] * jnp.take(x, r, axis=0), c,
                               num_segments=num_nodes)


def _layer(xr, xi, r, c, nr, ni, W, b, zeros):
    sigs = [xr, xi, xi, xr]
    nrms = [nr, ni, nr, ni]
    outs = [s @ W[0] for s in sigs]
    Tx1 = [_sc_prop(s, r, c, n, zeros) for s, n in zip(sigs, nrms)]
    outs = [o + t @ W[1] for o, t in zip(outs, Tx1)]
    Tx2 = [2.0 * _sc_prop(t1, r, c, n, zeros) - t0
           for t1, t0, n in zip(Tx1, sigs, nrms)]
    outs = [o + t @ W[2] for o, t in zip(outs, Tx2)]
    out_r = outs[0] - outs[1] + b
    out_i = outs[2] + outs[3] + b
    mask = (out_r >= 0).astype(out_r.dtype)
    return mask * out_r, mask * out_i


def _tail_kernel(z_ref, w_ref, b_ref, zn_ref, out_ref, pred_ref, prob_ref):
    z = z_ref[...]
    logits = jnp.dot(z, w_ref[...], preferred_element_type=jnp.float32)
    logits = logits + b_ref[...]
    m = jnp.max(logits, axis=1, keepdims=True)
    sh = logits - m
    ex = jnp.exp(sh)
    den = jnp.sum(ex, axis=1, keepdims=True)
    out_ref[...] = sh - jnp.log(den)
    prob_ref[...] = ex / den
    pred_ref[...] = jnp.broadcast_to(
        jnp.argmax(logits, axis=1, keepdims=True).astype(jnp.int32),
        logits.shape)
    nrm = jnp.maximum(jnp.sqrt(jnp.sum(z * z, axis=1, keepdims=True)), 1e-12)
    zn_ref[...] = z / nrm


def _tail(z, Wc, bc):
    n, f = z.shape
    l = Wc.shape[0]
    lp = 128
    wp = jnp.full((f, lp), 0.0, jnp.float32).at[:, :l].set(Wc.T)
    bp = jnp.full((1, lp), -1e30, jnp.float32).at[0, :l].set(bc)
    zn, out, pred, prob = pl.pallas_call(
        _tail_kernel,
        out_shape=[
            jax.ShapeDtypeStruct((n, f), jnp.float32),
            jax.ShapeDtypeStruct((n, lp), jnp.float32),
            jax.ShapeDtypeStruct((n, lp), jnp.int32),
            jax.ShapeDtypeStruct((n, lp), jnp.float32),
        ],
    )(z, wp, bp)
    return zn, out[:, :l], pred[:, 0], prob[:, :l]


def kernel(real, imag, edge_index, edge_weight, W1, b1, W2, b2, Wc, bc):
    num_nodes = real.shape[0]
    r, c, w_sym, theta = _coalesce(edge_index, edge_weight, num_nodes)
    nr, ni = _norms(r, c, w_sym, theta, num_nodes)
    pad = _E2P - r.shape[0]
    rp = jnp.pad(r, (0, pad))
    cp = jnp.pad(c, (0, pad))
    nrp = jnp.pad(nr, (0, pad))
    nip = jnp.pad(ni, (0, pad))
    zeros = jnp.zeros((_NPAD, real.shape[1]), jnp.float32)
    zeros_h = jnp.zeros((_NPAD, W2.shape[2]), jnp.float32)
    xr, xi = _layer(real, imag, rp, cp, nrp, nip, W1, b1, zeros)
    xr, xi = _layer(xr, xi, rp, cp, nrp, nip, W2, b2, zeros_h)
    z = jnp.concatenate([xr, xi], axis=-1)
    return _tail(z, Wc, bc)


# speedup vs baseline: 2.6490x; 1.5632x over previous
import functools

import jax
import jax.numpy as jnp
import numpy as np
from jax import lax
from jax.experimental import pallas as pl
from jax.experimental.pallas import tpu as pltpu
from jax.experimental.pallas import tpu_sc as plsc

_TWO_PI_Q = 2.0 * np.pi * 0.25

_N = 10000
_CH = 128
_NTILES = 32
_E2P = 643072
_PER_TILE = _E2P // _NTILES
_NCH = _PER_TILE // _CH
_NPAD = 10240
_ROWS_PER_TILE = _NPAD // 16


@functools.lru_cache(maxsize=None)
def _sc_prop_call(width):
    mesh = plsc.VectorSubcoreMesh(core_axis_name="c", subcore_axis_name="s")
    nvec = width // 16
    cparams = (None if width % 128 == 0
               else pltpu.CompilerParams(use_tc_tiling_on_sc=False))

    @functools.partial(
        pl.kernel,
        mesh=mesh,
        compiler_params=cparams,
        out_type=jax.ShapeDtypeStruct((2, _NPAD, width), jnp.float32),
        scratch_types=[
            pltpu.VMEM((_CH,), jnp.int32),
            pltpu.VMEM((_CH,), jnp.int32),
            pltpu.VMEM((_CH,), jnp.float32),
            pltpu.VMEM((_CH, width), jnp.float32),
            pltpu.VMEM((_CH, width), jnp.float32),
            pltpu.VMEM_SHARED((_NPAD, width), jnp.float32),
            pltpu.SemaphoreType.DMA,
        ],
    )
    def prop(tbl, ridx, cidx, nrm, zeros, out,
             idxr_v, idxc_v, nrm_v, rows_v, sc_v, accum, sem):
        cid = lax.axis_index("c")
        sid = lax.axis_index("s")
        rbase = sid * _ROWS_PER_TILE
        pltpu.sync_copy(zeros.at[pl.ds(rbase, _ROWS_PER_TILE)],
                        accum.at[pl.ds(rbase, _ROWS_PER_TILE)])
        plsc.subcore_barrier()
        ebase = (cid * 16 + sid) * _PER_TILE

        def chunk(k, carry):
            off = ebase + k * _CH
            pltpu.sync_copy(ridx.at[pl.ds(off, _CH)], idxr_v)
            pltpu.sync_copy(nrm.at[pl.ds(off, _CH)], nrm_v)
            pltpu.sync_copy(cidx.at[pl.ds(off, _CH)], idxc_v)
            pltpu.async_copy(tbl.at[idxr_v], rows_v, sem).wait()

            def grp(g, c2):
                n16 = nrm_v[pl.ds(g * 16, 16)]
                for lane in range(16):
                    i = g * 16 + lane
                    s = lax.gather(
                        n16, jnp.full((16, 1), lane, jnp.int32),
                        lax.GatherDimensionNumbers(
                            offset_dims=(), collapsed_slice_dims=(0,),
                            start_index_map=(0,)),
                        (1,), mode=lax.GatherScatterMode.PROMISE_IN_BOUNDS)
                    for j in range(nvec):
                        sc_v[i, pl.ds(j * 16, 16)] = (
                            rows_v[i, pl.ds(j * 16, 16)] * s)
                return c2

            lax.fori_loop(0, _CH // 16, grp, 0)
            pltpu.sync_copy(sc_v, accum.at[idxc_v], add=True)
            return carry

        lax.fori_loop(0, _NCH, chunk, 0)
        plsc.subcore_barrier()
        pltpu.sync_copy(accum.at[pl.ds(rbase, _ROWS_PER_TILE)],
                        out.at[cid, pl.ds(rbase, _ROWS_PER_TILE)])

    return prop


def _sc_prop(tbl, ridx, cidx, nrm, zeros):
    parts = _sc_prop_call(tbl.shape[1])(tbl, ridx, cidx, nrm, zeros)
    return parts[0, :_N] + parts[1, :_N]


@functools.lru_cache(maxsize=None)
def _sc_wn_call():
    mesh = plsc.VectorSubcoreMesh(core_axis_name="c", subcore_axis_name="s")

    @functools.partial(
        pl.kernel,
        mesh=mesh,
        compiler_params=pltpu.CompilerParams(use_tc_tiling_on_sc=False),
        out_type=jax.ShapeDtypeStruct((_E2P,), jnp.float32),
        scratch_types=[
            pltpu.VMEM((_CH,), jnp.int32),
            pltpu.VMEM((_CH,), jnp.int32),
            pltpu.VMEM((_CH,), jnp.float32),
            pltpu.VMEM((_CH,), jnp.float32),
            pltpu.VMEM((_CH, 16), jnp.float32),
            pltpu.VMEM((_CH, 16), jnp.float32),
            pltpu.SemaphoreType.DMA,
            pltpu.SemaphoreType.DMA,
        ],
    )
    def wn_kernel(dinv_t, ridx, cidx, wsym, out,
                  idxr_v, idxc_v, ws_v, out_v, rows_r, rows_c, sem, sem2):
        cid = lax.axis_index("c")
        sid = lax.axis_index("s")
        lane = lax.iota(jnp.int32, 16)
        ebase = (cid * 16 + sid) * _PER_TILE

        def chunk(k, carry):
            off = ebase + k * _CH
            pltpu.sync_copy(ridx.at[pl.ds(off, _CH)], idxr_v)
            pltpu.sync_copy(cidx.at[pl.ds(off, _CH)], idxc_v)
            pltpu.sync_copy(wsym.at[pl.ds(off, _CH)], ws_v)
            cp1 = pltpu.async_copy(dinv_t.at[idxr_v], rows_r, sem)
            cp2 = pltpu.async_copy(dinv_t.at[idxc_v], rows_c, sem2)
            cp1.wait()
            cp2.wait()

            def grp(g, c2):
                dr = jnp.zeros((16,), jnp.float32)
                dc = jnp.zeros((16,), jnp.float32)
                for e in range(16):
                    i = g * 16 + e
                    m = lane == e
                    dr = jnp.where(m, rows_r[i, pl.ds(0, 16)], dr)
                    dc = jnp.where(m, rows_c[i, pl.ds(0, 16)], dc)
                ws16 = ws_v[pl.ds(g * 16, 16)]
                out_v[pl.ds(g * 16, 16)] = -(dr * ws16 * dc)
                return c2

            lax.fori_loop(0, _CH // 16, grp, 0)
            pltpu.sync_copy(out_v, out.at[pl.ds(off, _CH)])
            return carry

        lax.fori_loop(0, _NCH, chunk, 0)

    return wn_kernel


def _coalesce(edge_index, edge_weight, num_nodes):
    r0 = edge_index[0].astype(jnp.int32)
    c0 = edge_index[1].astype(jnp.int32)
    ids = jnp.concatenate([r0 * num_nodes + c0, c0 * num_nodes + r0])
    w_both = jnp.concatenate([edge_weight, edge_weight])
    w_sgn = jnp.concatenate([edge_weight, -edge_weight])
    sk, sw, sd = jax.lax.sort([ids, w_both, w_sgn], num_keys=1)
    head = jnp.concatenate([jnp.ones((1,), jnp.bool_), sk[1:] != sk[:-1]])
    seg = jnp.cumsum(head.astype(jnp.int32)) - 1
    e2 = ids.shape[0]
    w_sym = jax.ops.segment_sum(sw, seg, num_segments=e2,
                                indices_are_sorted=True) * 0.5
    w_dif = jax.ops.segment_sum(sd, seg, num_segments=e2,
                                indices_are_sorted=True)
    key_u = jax.ops.segment_max(sk, seg, num_segments=e2,
                                indices_are_sorted=True)
    key_u = jnp.maximum(key_u, 0)
    r = key_u // num_nodes
    c = key_u % num_nodes
    return r, c, w_sym, w_dif * _TWO_PI_Q


def _norms(r, c, w_sym, theta, num_nodes):
    deg = jax.ops.segment_sum(jnp.abs(w_sym), r, num_segments=num_nodes)
    dinv = jnp.where(deg > 0, jax.lax.rsqrt(jnp.where(deg > 0, deg, 1.0)), 0.0)
    wn = -(dinv[r] * w_sym * dinv[c])
    return wn * jnp.cos(theta), wn * jnp.sin(theta)


def _prop(x, n, r, c, num_nodes):
    return jax.ops.segment_sum(n[:, None] * jnp.take(x, r, axis=0), c,
                               num_segments=num_nodes)


def _layer(xr, xi, r, c, nr, ni, W, b, zeros):
    sigs = [xr, xi, xi, xr]
    nrms = [nr, ni, nr, ni]
    outs = [s @ W[0] for s in sigs]
    Tx1 = [_sc_prop(s, r, c, n, zeros) for s, n in zip(sigs, nrms)]
    outs = [o + t @ W[1] for o, t in zip(outs, Tx1)]
    Tx2 = [2.0 * _sc_prop(t1, r, c, n, zeros) - t0
           for t1, t0, n in zip(Tx1, sigs, nrms)]
    outs = [o + t @ W[2] for o, t in zip(outs, Tx2)]
    out_r = outs[0] - outs[1] + b
    out_i = outs[2] + outs[3] + b
    mask = (out_r >= 0).astype(out_r.dtype)
    return mask * out_r, mask * out_i


def _tail_kernel(z_ref, w_ref, b_ref, zn_ref, out_ref, pred_ref, prob_ref):
    z = z_ref[...]
    logits = jnp.dot(z, w_ref[...], preferred_element_type=jnp.float32)
    logits = logits + b_ref[...]
    m = jnp.max(logits, axis=1, keepdims=True)
    sh = logits - m
    ex = jnp.exp(sh)
    den = jnp.sum(ex, axis=1, keepdims=True)
    out_ref[...] = sh - jnp.log(den)
    prob_ref[...] = ex / den
    pred_ref[...] = jnp.broadcast_to(
        jnp.argmax(logits, axis=1, keepdims=True).astype(jnp.int32),
        logits.shape)
    nrm = jnp.maximum(jnp.sqrt(jnp.sum(z * z, axis=1, keepdims=True)), 1e-12)
    zn_ref[...] = z / nrm


def _tail(z, Wc, bc):
    n, f = z.shape
    l = Wc.shape[0]
    lp = 128
    wp = jnp.full((f, lp), 0.0, jnp.float32).at[:, :l].set(Wc.T)
    bp = jnp.full((1, lp), -1e30, jnp.float32).at[0, :l].set(bc)
    zn, out, pred, prob = pl.pallas_call(
        _tail_kernel,
        out_shape=[
            jax.ShapeDtypeStruct((n, f), jnp.float32),
            jax.ShapeDtypeStruct((n, lp), jnp.float32),
            jax.ShapeDtypeStruct((n, lp), jnp.int32),
            jax.ShapeDtypeStruct((n, lp), jnp.float32),
        ],
    )(z, wp, bp)
    return zn, out[:, :l], pred[:, 0], prob[:, :l]


def kernel(real, imag, edge_index, edge_weight, W1, b1, W2, b2, Wc, bc):
    num_nodes = real.shape[0]
    r, c, w_sym, theta = _coalesce(edge_index, edge_weight, num_nodes)
    pad = _E2P - r.shape[0]
    rp = jnp.pad(r, (0, pad))
    cp = jnp.pad(c, (0, pad))
    wsp = jnp.pad(w_sym, (0, pad))
    thp = jnp.pad(theta, (0, pad))
    deg = jax.ops.segment_sum(jnp.abs(w_sym), r, num_segments=num_nodes)
    dinv = jnp.where(deg > 0, jax.lax.rsqrt(jnp.where(deg > 0, deg, 1.0)), 0.0)
    dinv_t = jnp.broadcast_to(dinv[:, None], (num_nodes, 16))
    wn = _sc_wn_call()(dinv_t, rp, cp, wsp)
    nrp = wn * jnp.cos(thp)
    nip = wn * jnp.sin(thp)
    zeros = jnp.zeros((_NPAD, real.shape[1]), jnp.float32)
    zeros_h = jnp.zeros((_NPAD, W2.shape[2]), jnp.float32)
    xr, xi = _layer(real, imag, rp, cp, nrp, nip, W1, b1, zeros)
    xr, xi = _layer(xr, xi, rp, cp, nrp, nip, W2, b2, zeros_h)
    z = jnp.concatenate([xr, xi], axis=-1)
    return _tail(z, Wc, bc)


# speedup vs baseline: 3.7846x; 1.4287x over previous
import functools

import jax
import jax.numpy as jnp
import numpy as np
from jax import lax
from jax.experimental import pallas as pl
from jax.experimental.pallas import tpu as pltpu
from jax.experimental.pallas import tpu_sc as plsc

_TWO_PI_Q = 2.0 * np.pi * 0.25

_N = 10000
_CH = 128
_NTILES = 32
_E2P = 647168
_PER_TILE = _E2P // _NTILES
_NCH = _PER_TILE // _CH
_NPAD = 10240
_ROWS_PER_TILE = _NPAD // 16


@functools.lru_cache(maxsize=None)
def _sc_prop_call(width):
    mesh = plsc.VectorSubcoreMesh(core_axis_name="c", subcore_axis_name="s")
    nvec = width // 16
    cparams = (None if width % 128 == 0
               else pltpu.CompilerParams(use_tc_tiling_on_sc=False))

    @functools.partial(
        pl.kernel,
        mesh=mesh,
        compiler_params=cparams,
        out_type=jax.ShapeDtypeStruct((2, _NPAD, width), jnp.float32),
        scratch_types=[
            pltpu.VMEM((_CH,), jnp.int32),
            pltpu.VMEM((_CH,), jnp.int32),
            pltpu.VMEM((_CH,), jnp.int32),
            pltpu.VMEM((_CH,), jnp.int32),
            pltpu.VMEM((_CH,), jnp.float32),
            pltpu.VMEM((_CH,), jnp.float32),
            pltpu.VMEM((_CH, width), jnp.float32),
            pltpu.VMEM((_CH, width), jnp.float32),
            pltpu.VMEM_SHARED((_NPAD, width), jnp.float32),
            pltpu.SemaphoreType.DMA,
            pltpu.SemaphoreType.DMA,
            pltpu.SemaphoreType.DMA,
            pltpu.SemaphoreType.DMA,
            pltpu.SemaphoreType.DMA,
            pltpu.SemaphoreType.DMA,
            pltpu.SemaphoreType.DMA,
            pltpu.SemaphoreType.DMA,
        ],
    )
    def prop(tbl, ridx, cidx, nrm, zeros, out,
             idxr0, idxr1, idxc0, idxc1, nrm0, nrm1,
             rows0, rows1, accum,
             sr0, sr1, scs0, scs1, sn0, sn1, sg0, sg1):
        cid = lax.axis_index("c")
        sid = lax.axis_index("s")
        rbase = sid * _ROWS_PER_TILE
        pltpu.sync_copy(zeros.at[pl.ds(rbase, _ROWS_PER_TILE)],
                        accum.at[pl.ds(rbase, _ROWS_PER_TILE)])
        plsc.subcore_barrier()
        ebase = (cid * 16 + sid) * _PER_TILE
        idxr = (idxr0, idxr1)
        idxc = (idxc0, idxc1)
        nrmv = (nrm0, nrm1)
        rows = (rows0, rows1)
        sr = (sr0, sr1)
        scs = (scs0, scs1)
        sn = (sn0, sn1)
        sg = (sg0, sg1)

        def fire_idx(k, b):
            off = ebase + k * _CH
            pltpu.async_copy(ridx.at[pl.ds(off, _CH)], idxr[b], sr[b])
            pltpu.async_copy(cidx.at[pl.ds(off, _CH)], idxc[b], scs[b])
            pltpu.async_copy(nrm.at[pl.ds(off, _CH)], nrmv[b], sn[b])

        def wait_idx(b):
            pltpu.make_async_copy(ridx.at[pl.ds(0, _CH)], idxr[b], sr[b]).wait()
            pltpu.make_async_copy(cidx.at[pl.ds(0, _CH)], idxc[b], scs[b]).wait()
            pltpu.make_async_copy(nrm.at[pl.ds(0, _CH)], nrmv[b], sn[b]).wait()

        def scale(b):
            def grp(g, c2):
                n16 = nrmv[b][pl.ds(g * 16, 16)]
                for lane in range(16):
                    i = g * 16 + lane
                    s = lax.gather(
                        n16, jnp.full((16, 1), lane, jnp.int32),
                        lax.GatherDimensionNumbers(
                            offset_dims=(), collapsed_slice_dims=(0,),
                            start_index_map=(0,)),
                        (1,), mode=lax.GatherScatterMode.PROMISE_IN_BOUNDS)
                    for j in range(nvec):
                        rows[b][i, pl.ds(j * 16, 16)] = (
                            rows[b][i, pl.ds(j * 16, 16)] * s)
                return c2

            lax.fori_loop(0, _CH // 16, grp, 0)

        fire_idx(0, 0)
        wait_idx(0)
        pltpu.async_copy(tbl.at[idxr[0]], rows[0], sg[0])
        fire_idx(1, 1)

        def body(j, carry):
            for b in range(2):
                k = j * 2 + b
                nb = 1 - b

                @pl.when(k + 1 < _NCH)
                def _():
                    wait_idx(nb)
                    pltpu.async_copy(tbl.at[idxr[nb]], rows[nb], sg[nb])

                pltpu.make_async_copy(
                    tbl.at[idxr[b]], rows[b], sg[b]).wait()
                scale(b)
                pltpu.sync_copy(rows[b], accum.at[idxc[b]], add=True)

                @pl.when(k + 2 < _NCH)
                def _():
                    fire_idx(k + 2, b)
            return carry

        lax.fori_loop(0, _NCH // 2, body, 0)
        plsc.subcore_barrier()
        pltpu.sync_copy(accum.at[pl.ds(rbase, _ROWS_PER_TILE)],
                        out.at[cid, pl.ds(rbase, _ROWS_PER_TILE)])

    return prop


def _sc_prop(tbl, ridx, cidx, nrm, zeros):
    parts = _sc_prop_call(tbl.shape[1])(tbl, ridx, cidx, nrm, zeros)
    return parts[0, :_N] + parts[1, :_N]


@functools.lru_cache(maxsize=None)
def _sc_wn_call():
    mesh = plsc.VectorSubcoreMesh(core_axis_name="c", subcore_axis_name="s")

    @functools.partial(
        pl.kernel,
        mesh=mesh,
        compiler_params=pltpu.CompilerParams(use_tc_tiling_on_sc=False),
        out_type=jax.ShapeDtypeStruct((_E2P,), jnp.float32),
        scratch_types=[
            pltpu.VMEM((_CH,), jnp.int32),
            pltpu.VMEM((_CH,), jnp.int32),
            pltpu.VMEM((_CH,), jnp.float32),
            pltpu.VMEM((_CH,), jnp.float32),
            pltpu.VMEM((_CH, 16), jnp.float32),
            pltpu.VMEM((_CH, 16), jnp.float32),
            pltpu.SemaphoreType.DMA,
            pltpu.SemaphoreType.DMA,
        ],
    )
    def wn_kernel(dinv_t, ridx, cidx, wsym, out,
                  idxr_v, idxc_v, ws_v, out_v, rows_r, rows_c, sem, sem2):
        cid = lax.axis_index("c")
        sid = lax.axis_index("s")
        lane = lax.iota(jnp.int32, 16)
        ebase = (cid * 16 + sid) * _PER_TILE

        def chunk(k, carry):
            off = ebase + k * _CH
            pltpu.sync_copy(ridx.at[pl.ds(off, _CH)], idxr_v)
            pltpu.sync_copy(cidx.at[pl.ds(off, _CH)], idxc_v)
            pltpu.sync_copy(wsym.at[pl.ds(off, _CH)], ws_v)
            cp1 = pltpu.async_copy(dinv_t.at[idxr_v], rows_r, sem)
            cp2 = pltpu.async_copy(dinv_t.at[idxc_v], rows_c, sem2)
            cp1.wait()
            cp2.wait()

            def grp(g, c2):
                dr = jnp.zeros((16,), jnp.float32)
                dc = jnp.zeros((16,), jnp.float32)
                for e in range(16):
                    i = g * 16 + e
                    m = lane == e
                    dr = jnp.where(m, rows_r[i, pl.ds(0, 16)], dr)
                    dc = jnp.where(m, rows_c[i, pl.ds(0, 16)], dc)
                ws16 = ws_v[pl.ds(g * 16, 16)]
                out_v[pl.ds(g * 16, 16)] = -(dr * ws16 * dc)
                return c2

            lax.fori_loop(0, _CH // 16, grp, 0)
            pltpu.sync_copy(out_v, out.at[pl.ds(off, _CH)])
            return carry

        lax.fori_loop(0, _NCH, chunk, 0)

    return wn_kernel


def _coalesce(edge_index, edge_weight, num_nodes):
    r0 = edge_index[0].astype(jnp.int32)
    c0 = edge_index[1].astype(jnp.int32)
    ids = jnp.concatenate([r0 * num_nodes + c0, c0 * num_nodes + r0])
    w_both = jnp.concatenate([edge_weight, edge_weight])
    w_sgn = jnp.concatenate([edge_weight, -edge_weight])
    sk, sw, sd = jax.lax.sort([ids, w_both, w_sgn], num_keys=1)
    head = jnp.concatenate([jnp.ones((1,), jnp.bool_), sk[1:] != sk[:-1]])
    seg = jnp.cumsum(head.astype(jnp.int32)) - 1
    e2 = ids.shape[0]
    w_sym = jax.ops.segment_sum(sw, seg, num_segments=e2,
                                indices_are_sorted=True) * 0.5
    w_dif = jax.ops.segment_sum(sd, seg, num_segments=e2,
                                indices_are_sorted=True)
    key_u = jax.ops.segment_max(sk, seg, num_segments=e2,
                                indices_are_sorted=True)
    key_u = jnp.maximum(key_u, 0)
    r = key_u // num_nodes
    c = key_u % num_nodes
    return r, c, w_sym, w_dif * _TWO_PI_Q


def _norms(r, c, w_sym, theta, num_nodes):
    deg = jax.ops.segment_sum(jnp.abs(w_sym), r, num_segments=num_nodes)
    dinv = jnp.where(deg > 0, jax.lax.rsqrt(jnp.where(deg > 0, deg, 1.0)), 0.0)
    wn = -(dinv[r] * w_sym * dinv[c])
    return wn * jnp.cos(theta), wn * jnp.sin(theta)


def _prop(x, n, r, c, num_nodes):
    return jax.ops.segment_sum(n[:, None] * jnp.take(x, r, axis=0), c,
                               num_segments=num_nodes)


def _layer(xr, xi, r, c, nr, ni, W, b, zeros):
    sigs = [xr, xi, xi, xr]
    nrms = [nr, ni, nr, ni]
    outs = [s @ W[0] for s in sigs]
    Tx1 = [_sc_prop(s, r, c, n, zeros) for s, n in zip(sigs, nrms)]
    outs = [o + t @ W[1] for o, t in zip(outs, Tx1)]
    Tx2 = [2.0 * _sc_prop(t1, r, c, n, zeros) - t0
           for t1, t0, n in zip(Tx1, sigs, nrms)]
    outs = [o + t @ W[2] for o, t in zip(outs, Tx2)]
    out_r = outs[0] - outs[1] + b
    out_i = outs[2] + outs[3] + b
    mask = (out_r >= 0).astype(out_r.dtype)
    return mask * out_r, mask * out_i


def _tail_kernel(z_ref, w_ref, b_ref, zn_ref, out_ref, pred_ref, prob_ref):
    z = z_ref[...]
    logits = jnp.dot(z, w_ref[...], preferred_element_type=jnp.float32)
    logits = logits + b_ref[...]
    m = jnp.max(logits, axis=1, keepdims=True)
    sh = logits - m
    ex = jnp.exp(sh)
    den = jnp.sum(ex, axis=1, keepdims=True)
    out_ref[...] = sh - jnp.log(den)
    prob_ref[...] = ex / den
    pred_ref[...] = jnp.broadcast_to(
        jnp.argmax(logits, axis=1, keepdims=True).astype(jnp.int32),
        logits.shape)
    nrm = jnp.maximum(jnp.sqrt(jnp.sum(z * z, axis=1, keepdims=True)), 1e-12)
    zn_ref[...] = z / nrm


def _tail(z, Wc, bc):
    n, f = z.shape
    l = Wc.shape[0]
    lp = 128
    wp = jnp.full((f, lp), 0.0, jnp.float32).at[:, :l].set(Wc.T)
    bp = jnp.full((1, lp), -1e30, jnp.float32).at[0, :l].set(bc)
    zn, out, pred, prob = pl.pallas_call(
        _tail_kernel,
        out_shape=[
            jax.ShapeDtypeStruct((n, f), jnp.float32),
            jax.ShapeDtypeStruct((n, lp), jnp.float32),
            jax.ShapeDtypeStruct((n, lp), jnp.int32),
            jax.ShapeDtypeStruct((n, lp), jnp.float32),
        ],
    )(z, wp, bp)
    return zn, out[:, :l], pred[:, 0], prob[:, :l]


def kernel(real, imag, edge_index, edge_weight, W1, b1, W2, b2, Wc, bc):
    num_nodes = real.shape[0]
    r, c, w_sym, theta = _coalesce(edge_index, edge_weight, num_nodes)
    pad = _E2P - r.shape[0]
    rp = jnp.pad(r, (0, pad))
    cp = jnp.pad(c, (0, pad))
    wsp = jnp.pad(w_sym, (0, pad))
    thp = jnp.pad(theta, (0, pad))
    deg = jax.ops.segment_sum(jnp.abs(w_sym), r, num_segments=num_nodes)
    dinv = jnp.where(deg > 0, jax.lax.rsqrt(jnp.where(deg > 0, deg, 1.0)), 0.0)
    dinv_t = jnp.broadcast_to(dinv[:, None], (num_nodes, 16))
    wn = _sc_wn_call()(dinv_t, rp, cp, wsp)
    nrp = wn * jnp.cos(thp)
    nip = wn * jnp.sin(thp)
    zeros = jnp.zeros((_NPAD, real.shape[1]), jnp.float32)
    zeros_h = jnp.zeros((_NPAD, W2.shape[2]), jnp.float32)
    xr, xi = _layer(real, imag, rp, cp, nrp, nip, W1, b1, zeros)
    xr, xi = _layer(xr, xi, rp, cp, nrp, nip, W2, b2, zeros_h)
    z = jnp.concatenate([xr, xi], axis=-1)
    return _tail(z, Wc, bc)
